# baseline (speedup 1.0000x reference)
import functools

import jax
import jax.numpy as jnp
from jax import lax
from jax.experimental import pallas as pl
from jax.experimental.pallas import tpu as pltpu
from jax.experimental.pallas import tpu_sc as plsc

N = 10000
E = 320000
D = 128
DH = D // 2

NC = 2
NS = 16
NW = NC * NS

NPAD = 10240
LDEG = 2 * NPAD
RED = LDEG // NS

C = 80
EPT = E // NS
CH = EPT // C
NBUF = 4
SPAN = 624
SCH = 208

_mesh = plsc.VectorSubcoreMesh(core_axis_name="c", subcore_axis_name="s")


def _zero_zbuf(zbuf):
    zeros = jnp.zeros((16,), jnp.float32)

    def zvbody(i, _):
        for k in range(DH // 16):
            zbuf[i, pl.ds(k * 16, 16)] = zeros
        return 0
    lax.fori_loop(0, 16, zvbody, 0)


def _zero_agg(zbuf, agg, s):
    def zbody(k, _):
        pltpu.sync_copy(zbuf, agg.at[pl.ds(s * SPAN + k * 16, 16)])
        return 0
    lax.fori_loop(0, SPAN // 16, zbody, 0)

    @pl.when(s == NS - 1)
    def _():
        pltpu.sync_copy(zbuf, agg.at[pl.ds(NS * SPAN, 16)])


def _edge_phase(hsrc, visrc, vidst, rows, agg, gsem, ssem):
    pltpu.async_copy(hsrc.at[visrc.at[0]], rows.at[0], gsem.at[0])
    pltpu.async_copy(hsrc.at[visrc.at[1]], rows.at[1], gsem.at[1])

    def body(j, _):
        b = lax.rem(j, NBUF)
        bn = lax.rem(j + 2, NBUF)

        @pl.when(j >= 2)
        def _():
            pltpu.make_async_copy(rows.at[bn], agg.at[vidst.at[j - 2]],
                                  ssem.at[bn]).wait()

        @pl.when(j + 2 < CH)
        def _():
            pltpu.async_copy(hsrc.at[visrc.at[j + 2]], rows.at[bn],
                             gsem.at[bn])

        pltpu.make_async_copy(hsrc.at[visrc.at[j]], rows.at[b],
                              gsem.at[b]).wait()
        pltpu.async_copy(rows.at[b], agg.at[vidst.at[j]], ssem.at[b],
                         add=True)
        return 0
    lax.fori_loop(0, CH, body, 0)

    for j in (CH - 2, CH - 1):
        b = j % NBUF
        pltpu.make_async_copy(rows.at[b], agg.at[vidst.at[j]],
                              ssem.at[b]).wait()


def _writeout(agg, part, c, s):
    pltpu.sync_copy(agg.at[pl.ds(s * SPAN, SPAN)],
                    part.at[c, pl.ds(s * SPAN, SPAN)])

    @pl.when(s == NS - 1)
    def _():
        pltpu.sync_copy(agg.at[pl.ds(NS * SPAN, 16)],
                        part.at[c, pl.ds(NS * SPAN, 16)])


def _rsqrt16(d):
    i = plsc.bitcast(d, jnp.int32)
    y = plsc.bitcast(jnp.int32(0x5F3759DF) - (i >> 1), jnp.float32)
    for _ in range(3):
        y = y * (1.5 - 0.5 * d * y * y)
    return y


def _l1_body(xw, srcI, dstI, h1, degs,
             visrc, vidst, ldeg, rbuf, obuf, dbuf, nbuf, sbuf,
             slots, deg_sh):
    c = lax.axis_index("c")
    s = lax.axis_index("s")
    pltpu.sync_copy(srcI.at[s], visrc)
    pltpu.sync_copy(dstI.at[s], vidst)

    zeros = jnp.zeros((16,), jnp.float32)
    ones = jnp.ones((16,), jnp.float32)

    def zlbody(i, _):
        ldeg[pl.ds(i * 16, 16)] = zeros
        return 0
    lax.fori_loop(0, LDEG // 16, zlbody, 0)

    def hbody(r, _):
        for k in range(C // 16):
            plsc.addupdate_scatter(ldeg, [visrc[r, pl.ds(k * 16, 16)]], ones)
            plsc.addupdate_scatter(
                ldeg, [vidst[r, pl.ds(k * 16, 16)] + NPAD], ones)
        return 0
    lax.fori_loop(0, CH, hbody, 0)

    pltpu.sync_copy(ldeg, slots.at[pl.ds(s * LDEG, LDEG)])
    plsc.subcore_barrier()

    for t in range(NS):
        pltpu.sync_copy(slots.at[pl.ds(t * LDEG + s * RED, RED)],
                        rbuf.at[pl.ds(t * RED, RED)])

    def rbody(v, _):
        acc = rbuf[pl.ds(v * 16, 16)]
        for t in range(1, NS):
            acc = acc + rbuf[pl.ds(t * RED + v * 16, 16)]
        obuf[pl.ds(v * 16, 16)] = acc
        return 0
    lax.fori_loop(0, RED // 16, rbody, 0)
    pltpu.sync_copy(obuf, deg_sh.at[pl.ds(s * RED, RED)])
    pltpu.sync_copy(obuf, degs.at[c, 0, pl.ds(s * RED, RED)])
    plsc.subcore_barrier()

    pltpu.sync_copy(deg_sh.at[pl.ds(s * SPAN, 640)], dbuf)

    def nbody(v, _):
        d = jnp.maximum(dbuf[pl.ds(v * 16, 16)], 1.0)
        nbuf[pl.ds(v * 16, 16)] = _rsqrt16(d)
        return 0
    lax.fori_loop(0, 640 // 16, nbody, 0)

    def scale_rows(nrows, chunk_base):
        pltpu.sync_copy(
            xw.at[pl.ds(s * SPAN + chunk_base, nrows), pl.ds(c * DH, DH)],
            sbuf.at[pl.ds(0, nrows)])

        def srow(g, _):
            nv = nbuf[pl.ds(chunk_base + g * 16, 16)]
            for l in range(16):
                sv = jnp.full((16,), nv[l], jnp.float32)
                r = g * 16 + l
                for k in range(DH // 16):
                    sbuf[r, pl.ds(k * 16, 16)] = (
                        sbuf[r, pl.ds(k * 16, 16)] * sv)
            return 0
        lax.fori_loop(0, nrows // 16, srow, 0)
        pltpu.sync_copy(
            sbuf.at[pl.ds(0, nrows)],
            h1.at[c, pl.ds(s * SPAN + chunk_base, nrows)])

    for q in range(SPAN // SCH):
        scale_rows(SCH, q * SCH)

    @pl.when(s == NS - 1)
    def _():
        scale_rows(16, SPAN)


_l1_kernel = functools.partial(
    pl.kernel,
    out_type=(
        jax.ShapeDtypeStruct((NC, N, DH), jnp.float32),
        jax.ShapeDtypeStruct((NC, 1, LDEG), jnp.float32),
    ),
    mesh=_mesh,
    scratch_types=[
        pltpu.VMEM((CH, C), jnp.int32),
        pltpu.VMEM((CH, C), jnp.int32),
        pltpu.VMEM((LDEG,), jnp.float32),
        pltpu.VMEM((NS * RED,), jnp.float32),
        pltpu.VMEM((RED,), jnp.float32),
        pltpu.VMEM((640,), jnp.float32),
        pltpu.VMEM((640,), jnp.float32),
        pltpu.VMEM((SCH, DH), jnp.float32),
        pltpu.VMEM_SHARED((NS * LDEG,), jnp.float32),
        pltpu.VMEM_SHARED((LDEG,), jnp.float32),
    ],
    compiler_params=pltpu.CompilerParams(
        needs_layout_passes=False, use_tc_tiling_on_sc=False),
)(_l1_body)


def _edge_body(hp, srcI, dstI, part, visrc, vidst, rows, zbuf, agg,
               gsem, ssem):
    c = lax.axis_index("c")
    s = lax.axis_index("s")
    pltpu.sync_copy(srcI.at[s], visrc)
    pltpu.sync_copy(dstI.at[s], vidst)

    _zero_zbuf(zbuf)
    _zero_agg(zbuf, agg, s)
    plsc.subcore_barrier()

    _edge_phase(hp.at[c], visrc, vidst, rows, agg, gsem, ssem)

    plsc.subcore_barrier()
    _writeout(agg, part, c, s)


_edge_kernel = functools.partial(
    pl.kernel,
    out_type=jax.ShapeDtypeStruct((NC, N, DH), jnp.float32),
    mesh=_mesh,
    scratch_types=[
        pltpu.VMEM((CH, C), jnp.int32),
        pltpu.VMEM((CH, C), jnp.int32),
        pltpu.VMEM((NBUF, C, DH), jnp.float32),
        pltpu.VMEM((16, DH), jnp.float32),
        pltpu.VMEM_SHARED((N, DH), jnp.float32),
        pltpu.SemaphoreType.DMA((NBUF,)),
        pltpu.SemaphoreType.DMA((NBUF,)),
    ],
    compiler_params=pltpu.CompilerParams(
        needs_layout_passes=False, use_tc_tiling_on_sc=False),
)(_edge_body)


BR = 400
GRID = N // BR


def _mm0_body(x_ref, w_ref, o_ref):
    o_ref[...] = jnp.dot(x_ref[...], w_ref[...],
                         preferred_element_type=jnp.float32)


def _mid_body(p_ref, ddst_ref, b_ref, w_ref, dsrc_ref, o_ref):
    ndst = lax.rsqrt(jnp.maximum(ddst_ref[...], 1.0))
    nsrc = lax.rsqrt(jnp.maximum(dsrc_ref[...], 1.0))
    agg = jnp.concatenate([p_ref[0], p_ref[1]], axis=-1) * ndst + b_ref[...]
    hmid = jnp.maximum(agg, 0.0)
    res = jnp.dot(hmid, w_ref[...],
                  preferred_element_type=jnp.float32) * nsrc
    o_ref[0] = res[:, :DH]
    o_ref[1] = res[:, DH:]


def _fin_body(p_ref, ddst_ref, b_ref, g_ref, bt_ref, o_ref):
    ndst = lax.rsqrt(jnp.maximum(ddst_ref[...], 1.0))
    agg = jnp.concatenate([p_ref[0], p_ref[1]], axis=-1) * ndst + b_ref[...]
    mu = jnp.mean(agg, axis=-1, keepdims=True)
    dvar = agg - mu
    var = jnp.mean(dvar * dvar, axis=-1, keepdims=True)
    o_ref[...] = dvar * lax.rsqrt(var + 1e-5) * g_ref[...] + bt_ref[...]


_row_spec = pl.BlockSpec((BR, D), lambda i: (i, 0))
_col_spec = pl.BlockSpec((BR, 1), lambda i: (i, 0))
_full_spec = pl.BlockSpec((1, D), lambda i: (0, 0))
_w_spec = pl.BlockSpec((D, D), lambda i: (0, 0))
_p_spec = pl.BlockSpec((NC, BR, DH), lambda i: (0, i, 0))
_out_f32 = jax.ShapeDtypeStruct((N, D), jnp.float32)
_hp_shape = jax.ShapeDtypeStruct((NC, N, DH), jnp.float32)

_mm0 = pl.pallas_call(
    _mm0_body, grid=(GRID,),
    in_specs=[_row_spec, _w_spec],
    out_specs=_row_spec, out_shape=_out_f32)

_mid = pl.pallas_call(
    _mid_body, grid=(GRID,),
    in_specs=[_p_spec, _col_spec, _full_spec, _w_spec, _col_spec],
    out_specs=_p_spec, out_shape=_hp_shape)

_fin = pl.pallas_call(
    _fin_body, grid=(GRID,),
    in_specs=[_p_spec, _col_spec, _full_spec, _full_spec, _full_spec],
    out_specs=_row_spec, out_shape=_out_f32)


def kernel(x, edge_index, W1, b1, W2, b2, ln_gamma, ln_beta):
    src = edge_index[0]
    dst = edge_index[1]
    srcI = src.reshape(NS, CH, C)
    dstI = dst.reshape(NS, CH, C)

    xw = _mm0(x, W1)
    h1, degs3 = _l1_kernel(xw, srcI, dstI)
    p1 = _edge_kernel(h1, srcI, dstI)
    degs = degs3[0, 0]
    dsrc = degs[:N].reshape(N, 1)
    ddst = degs[NPAD:NPAD + N].reshape(N, 1)

    b1r = b1.reshape(1, D)
    b2r = b2.reshape(1, D)
    gr = ln_gamma.reshape(1, D)
    btr = ln_beta.reshape(1, D)

    h2 = _mid(p1, ddst, b1r, W2, dsrc)
    p2 = _edge_kernel(h2, srcI, dstI)
    return _fin(p2, ddst, b2r, gr, btr)


# speedup vs baseline: 1.1140x; 1.1140x over previous
import functools

import jax
import jax.numpy as jnp
from jax import lax
from jax.experimental import pallas as pl
from jax.experimental.pallas import tpu as pltpu
from jax.experimental.pallas import tpu_sc as plsc

N = 10000
E = 320000
D = 128
DH = D // 2

NC = 2
NS = 16
NW = NC * NS

NPAD = 10240
LDEG = 2 * NPAD
EPW = E // NW
DEG_ROWS = EPW // 16
RED = LDEG // NS

C = 80
EPT = E // NS
CH = EPT // C
NBUF = 6
SPAN = 624

_mesh = plsc.VectorSubcoreMesh(core_axis_name="c", subcore_axis_name="s")


def _deg_body(src16, dst16, degs, vsrc, vdst, ldeg, rbuf, obuf, slots):
    c = lax.axis_index("c")
    s = lax.axis_index("s")
    wid = c * NS + s
    pltpu.sync_copy(src16.at[wid], vsrc)
    pltpu.sync_copy(dst16.at[wid], vdst)

    zeros = jnp.zeros((16,), jnp.float32)
    ones = jnp.ones((16,), jnp.float32)

    def zbody(i, _):
        ldeg[pl.ds(i * 16, 16)] = zeros
        return 0
    lax.fori_loop(0, LDEG // 16, zbody, 0)

    def hbody(j, _):
        plsc.addupdate_scatter(ldeg, [vsrc[j]], ones)
        plsc.addupdate_scatter(ldeg, [vdst[j] + NPAD], ones)
        return 0
    lax.fori_loop(0, DEG_ROWS, hbody, 0)

    pltpu.sync_copy(ldeg, slots.at[pl.ds(s * LDEG, LDEG)])
    plsc.subcore_barrier()
    for t in range(NS):
        pltpu.sync_copy(slots.at[pl.ds(t * LDEG + s * RED, RED)],
                        rbuf.at[pl.ds(t * RED, RED)])

    def rbody(v, _):
        acc = rbuf[pl.ds(v * 16, 16)]
        for t in range(1, NS):
            acc = acc + rbuf[pl.ds(t * RED + v * 16, 16)]
        obuf[pl.ds(v * 16, 16)] = acc
        return 0
    lax.fori_loop(0, RED // 16, rbody, 0)
    pltpu.sync_copy(obuf, degs.at[c, 0, pl.ds(s * RED, RED)])


_deg_kernel = functools.partial(
    pl.kernel,
    out_type=jax.ShapeDtypeStruct((NC, 1, LDEG), jnp.float32),
    mesh=_mesh,
    scratch_types=[
        pltpu.VMEM((DEG_ROWS, 16), jnp.int32),
        pltpu.VMEM((DEG_ROWS, 16), jnp.int32),
        pltpu.VMEM((LDEG,), jnp.float32),
        pltpu.VMEM((NS * RED,), jnp.float32),
        pltpu.VMEM((RED,), jnp.float32),
        pltpu.VMEM_SHARED((NS * LDEG,), jnp.float32),
    ],
    compiler_params=pltpu.CompilerParams(
        needs_layout_passes=False, use_tc_tiling_on_sc=False),
)(_deg_body)


def _edge_body(hp, srcI, dstI, part, visrc, vidst, rows, zbuf, agg,
               gsem, ssem):
    c = lax.axis_index("c")
    s = lax.axis_index("s")
    pltpu.sync_copy(srcI.at[s], visrc)
    pltpu.sync_copy(dstI.at[s], vidst)

    zeros = jnp.zeros((16,), jnp.float32)

    def zvbody(i, _):
        for k in range(DH // 16):
            zbuf[i, pl.ds(k * 16, 16)] = zeros
        return 0
    lax.fori_loop(0, 16, zvbody, 0)

    def zbody(k, _):
        pltpu.sync_copy(zbuf, agg.at[pl.ds(s * SPAN + k * 16, 16)])
        return 0
    lax.fori_loop(0, SPAN // 16, zbody, 0)

    @pl.when(s == NS - 1)
    def _():
        pltpu.sync_copy(zbuf, agg.at[pl.ds(NS * SPAN, 16)])

    plsc.subcore_barrier()

    hsrc = hp.at[c]
    pltpu.async_copy(hsrc.at[visrc.at[0]], rows.at[0], gsem.at[0])
    pltpu.async_copy(hsrc.at[visrc.at[1]], rows.at[1], gsem.at[1])
    pltpu.async_copy(hsrc.at[visrc.at[2]], rows.at[2], gsem.at[2])

    def body(j, _):
        b = lax.rem(j, NBUF)
        bn = lax.rem(j + 3, NBUF)

        @pl.when(j >= 3)
        def _():
            pltpu.make_async_copy(rows.at[bn], agg.at[vidst.at[j - 3]],
                                  ssem.at[bn]).wait()

        @pl.when(j + 3 < CH)
        def _():
            pltpu.async_copy(hsrc.at[visrc.at[j + 3]], rows.at[bn],
                             gsem.at[bn])

        pltpu.make_async_copy(hsrc.at[visrc.at[j]], rows.at[b],
                              gsem.at[b]).wait()
        pltpu.async_copy(rows.at[b], agg.at[vidst.at[j]], ssem.at[b],
                         add=True)
        return 0
    lax.fori_loop(0, CH, body, 0)

    for j in (CH - 3, CH - 2, CH - 1):
        b = j % NBUF
        pltpu.make_async_copy(rows.at[b], agg.at[vidst.at[j]],
                              ssem.at[b]).wait()

    plsc.subcore_barrier()
    pltpu.sync_copy(agg.at[pl.ds(s * SPAN, SPAN)],
                    part.at[c, pl.ds(s * SPAN, SPAN)])

    @pl.when(s == NS - 1)
    def _():
        pltpu.sync_copy(agg.at[pl.ds(NS * SPAN, 16)],
                        part.at[c, pl.ds(NS * SPAN, 16)])


_edge_kernel = functools.partial(
    pl.kernel,
    out_type=jax.ShapeDtypeStruct((NC, N, DH), jnp.float32),
    mesh=_mesh,
    scratch_types=[
        pltpu.VMEM((CH, C), jnp.int32),
        pltpu.VMEM((CH, C), jnp.int32),
        pltpu.VMEM((NBUF, C, DH), jnp.float32),
        pltpu.VMEM((16, DH), jnp.float32),
        pltpu.VMEM_SHARED((N, DH), jnp.float32),
        pltpu.SemaphoreType.DMA((NBUF,)),
        pltpu.SemaphoreType.DMA((NBUF,)),
    ],
    compiler_params=pltpu.CompilerParams(
        needs_layout_passes=False, use_tc_tiling_on_sc=False),
)(_edge_body)


BR = 400
GRID = N // BR


def _mm1_body(x_ref, w_ref, dsrc_ref, o_ref):
    nsrc = lax.rsqrt(jnp.maximum(dsrc_ref[...], 1.0))
    res = jnp.dot(x_ref[...], w_ref[...],
                  preferred_element_type=jnp.float32) * nsrc
    o_ref[0] = res[:, :DH]
    o_ref[1] = res[:, DH:]


def _mid_body(p_ref, ddst_ref, b_ref, w_ref, dsrc_ref, o_ref):
    ndst = lax.rsqrt(jnp.maximum(ddst_ref[...], 1.0))
    nsrc = lax.rsqrt(jnp.maximum(dsrc_ref[...], 1.0))
    agg = jnp.concatenate([p_ref[0], p_ref[1]], axis=-1) * ndst + b_ref[...]
    hmid = jnp.maximum(agg, 0.0)
    res = jnp.dot(hmid, w_ref[...],
                  preferred_element_type=jnp.float32) * nsrc
    o_ref[0] = res[:, :DH]
    o_ref[1] = res[:, DH:]


def _fin_body(p_ref, ddst_ref, b_ref, g_ref, bt_ref, o_ref):
    ndst = lax.rsqrt(jnp.maximum(ddst_ref[...], 1.0))
    agg = jnp.concatenate([p_ref[0], p_ref[1]], axis=-1) * ndst + b_ref[...]
    mu = jnp.mean(agg, axis=-1, keepdims=True)
    dvar = agg - mu
    var = jnp.mean(dvar * dvar, axis=-1, keepdims=True)
    o_ref[...] = dvar * lax.rsqrt(var + 1e-5) * g_ref[...] + bt_ref[...]


_row_spec = pl.BlockSpec((BR, D), lambda i: (i, 0))
_col_spec = pl.BlockSpec((BR, 1), lambda i: (i, 0))
_full_spec = pl.BlockSpec((1, D), lambda i: (0, 0))
_w_spec = pl.BlockSpec((D, D), lambda i: (0, 0))
_p_spec = pl.BlockSpec((NC, BR, DH), lambda i: (0, i, 0))
_out_f32 = jax.ShapeDtypeStruct((N, D), jnp.float32)
_hp_shape = jax.ShapeDtypeStruct((NC, N, DH), jnp.float32)

_mm1 = pl.pallas_call(
    _mm1_body, grid=(GRID,),
    in_specs=[_row_spec, _w_spec, _col_spec],
    out_specs=_p_spec, out_shape=_hp_shape)

_mid = pl.pallas_call(
    _mid_body, grid=(GRID,),
    in_specs=[_p_spec, _col_spec, _full_spec, _w_spec, _col_spec],
    out_specs=_p_spec, out_shape=_hp_shape)

_fin = pl.pallas_call(
    _fin_body, grid=(GRID,),
    in_specs=[_p_spec, _col_spec, _full_spec, _full_spec, _full_spec],
    out_specs=_row_spec, out_shape=_out_f32)


def kernel(x, edge_index, W1, b1, W2, b2, ln_gamma, ln_beta):
    src = edge_index[0]
    dst = edge_index[1]
    src16 = src.reshape(NW, DEG_ROWS, 16)
    dst16 = dst.reshape(NW, DEG_ROWS, 16)
    srcI = src.reshape(NS, CH, C)
    dstI = dst.reshape(NS, CH, C)

    degs = _deg_kernel(src16, dst16)[:, 0]
    dsrc = (degs[0, :N] + degs[1, :N]).reshape(N, 1)
    ddst = (degs[0, NPAD:NPAD + N] + degs[1, NPAD:NPAD + N]).reshape(N, 1)

    b1r = b1.reshape(1, D)
    b2r = b2.reshape(1, D)
    gr = ln_gamma.reshape(1, D)
    btr = ln_beta.reshape(1, D)

    h1 = _mm1(x, W1, dsrc)
    p1 = _edge_kernel(h1, srcI, dstI)
    h2 = _mid(p1, ddst, b1r, W2, dsrc)
    p2 = _edge_kernel(h2, srcI, dstI)
    return _fin(p2, ddst, b2r, gr, btr)
